# Initial kernel scaffold; baseline (speedup 1.0000x reference)
import jax
import jax.numpy as jnp
from jax.experimental import pallas as pl


def kernel(observed_pose, pred_pose):
    raise NotImplementedError("write your pallas kernel here")


# speedup vs baseline: 1.3437x; 1.3437x over previous
import numpy as np
import jax
import jax.numpy as jnp
from jax.experimental import pallas as pl

_DIM_USED = np.array([6, 7, 8, 9, 10, 11, 12, 13, 14, 15, 16, 17, 21, 22, 23,
                      24, 25, 26, 27, 28, 29, 30, 31, 32, 36, 37, 38, 39, 40,
                      41, 42, 43, 44, 45, 46, 47, 51, 52, 53, 54, 55, 56, 57,
                      58, 59, 63, 64, 65, 66, 67, 68, 75, 76, 77, 78, 79, 80,
                      81, 82, 83, 87, 88, 89, 90, 91, 92])


def _j2i(x):
    return np.concatenate((x * 3, x * 3 + 1, x * 3 + 2))


_IDX_COPY = _j2i(np.array([0, 1, 6, 11]))
_IDX_EQUAL = _j2i(np.array([13, 19, 22, 13, 27, 30]))
_IDX_IGNORE = _j2i(np.array([16, 20, 23, 24, 28, 31]))

_MEAN = np.array([-108.0207, -357.1349, 162.4628, -60.3845, -749.3369, 54.4319, -62.5515, -800.5543, 189.4915, -72.9842, -804.0154, 254.9969, 150.332, -345.0031, 167.558, 133.2313, -745.717, 69.9487, 150.413, -782.4758, 205.3246, 159.4026, -775.569, 269.4921, -17.8942, 222.0518, -13.1888, -30.9244, 453.2491, 38.2384, -42.9791, 515.5094, 121.5481, -54.5557, 609.9525, 92.0117, 108.75, 415.4085, 26.6593, 236.89, 221.6402, 45.1059, 186.8823, 145.6833, 149.6532, 151.0743, 178.7816, 164.5888, 195.3418, 129.4365, 194.8929, -164.118, 396.0086, 24.8246, -265.7541, 195.1504, 54.709, -215.6535, 165.0968, 169.1314, -198.0947, 202.279, 171.9553, -220.7997, 158.5216, 222.7706], dtype=np.float64)

_STD = np.array([66.3929, 119.1274, 151.6286, 112.3741, 175.5857, 208.0643, 125.001, 168.7289, 222.087, 132.8225, 168.7699, 224.09, 75.9225, 136.5678, 156.0372, 132.4222, 182.5029, 208.6887, 158.7869, 174.4053, 218.7376, 170.3442, 173.4497, 219.1369, 19.7102, 17.9596, 63.2551, 52.4272, 51.2776, 137.7591, 68.3521, 84.5895, 155.2936, 78.1808, 85.8814, 186.8668, 47.0589, 52.5975, 136.8256, 72.152, 107.0486, 169.2219, 151.7592, 199.0935, 191.9489, 155.2793, 205.1654, 185.0615, 190.2583, 245.5722, 214.5396, 44.9131, 56.8072, 131.6514, 73.948, 119.211, 158.9703, 141.729, 228.1124, 186.8918, 146.0417, 220.1406, 180.3148, 185.631, 302.1246, 223.1294], dtype=np.float64)


def _build_maps():
    pos = {int(d): i for i, d in enumerate(_DIM_USED)}
    g = np.zeros(96, dtype=np.int32)
    scale = np.zeros(96, dtype=np.float64)
    bias = np.zeros(96, dtype=np.float64)
    cmask = np.zeros(96, dtype=np.float64)
    for i, d in enumerate(_DIM_USED):
        g[d] = i
        scale[d] = _STD[i]
        bias[d] = _MEAN[i]
    for i, d in enumerate(_IDX_IGNORE):
        j = pos[int(_IDX_EQUAL[i])]
        g[d] = j
        scale[d] = _STD[j]
        bias[d] = _MEAN[j]
    for d in _IDX_COPY:
        g[d] = 0
        scale[d] = 0.0
        bias[d] = 0.0
        cmask[d] = 1.0
    return g, scale, bias, cmask


_G96, _SCALE96, _BIAS96, _CMASK96 = _build_maps()

_W = np.zeros((66, 96), dtype=np.float64)
_W[_G96, np.arange(96)] = _SCALE96
_W_f32 = jnp.asarray(_W, dtype=jnp.float32)
_BIAS_f32 = jnp.asarray(_BIAS96.reshape(1, 96), dtype=jnp.float32)
_CMASK_f32 = jnp.asarray(_CMASK96.reshape(1, 96), dtype=jnp.float32)

_BB = 256


def _tc_body(w_ref, bias_ref, cmask_ref, obs_ref, pred_ref, out_ref):
    w = w_ref[...]
    b = bias_ref[...]
    o = obs_ref[...] * cmask_ref[...]
    for t in range(25):
        p_t = pred_ref[:, t, :]
        y = jax.lax.dot_general(
            p_t, w, (((1,), (0,)), ((), ())),
            preferred_element_type=jnp.float32,
            precision=jax.lax.Precision.HIGHEST)
        out_ref[:, t, :] = y + b + o


def kernel(observed_pose, pred_pose):
    B, T, _ = pred_pose.shape
    obs_last = observed_pose[:, -1, :]
    grid = (B // _BB,)
    return pl.pallas_call(
        _tc_body,
        grid=grid,
        in_specs=[
            pl.BlockSpec((66, 96), lambda i: (0, 0)),
            pl.BlockSpec((1, 96), lambda i: (0, 0)),
            pl.BlockSpec((1, 96), lambda i: (0, 0)),
            pl.BlockSpec((_BB, 96), lambda i: (i, 0)),
            pl.BlockSpec((_BB, T, 66), lambda i: (i, 0, 0)),
        ],
        out_specs=pl.BlockSpec((_BB, T, 96), lambda i: (i, 0, 0)),
        out_shape=jax.ShapeDtypeStruct((B, T, 96), jnp.float32),
    )(_W_f32, _BIAS_f32, _CMASK_f32, obs_last, pred_pose)
